# Initial kernel scaffold; baseline (speedup 1.0000x reference)
import jax
import jax.numpy as jnp
from jax.experimental import pallas as pl


def kernel(x, table):
    raise NotImplementedError("write your pallas kernel here")


# speedup vs baseline: 2.5136x; 2.5136x over previous
import functools

import jax
import jax.numpy as jnp
from jax import lax
from jax.experimental import pallas as pl
from jax.experimental.pallas import tpu as pltpu
from jax.experimental.pallas import tpu_sc as plsc

_B = 4
_L = 4096
_H = 1024
_CH = 64


@jax.jit
def _broadcast_rows(table):
    info = plsc.get_sparse_core_info()
    num_workers = info.num_cores * info.num_subcores
    rows_per_w = _L // num_workers
    n_ch = rows_per_w // _CH
    mesh = plsc.VectorSubcoreMesh(core_axis_name="c", subcore_axis_name="s")

    @functools.partial(
        pl.kernel,
        mesh=mesh,
        out_type=jax.ShapeDtypeStruct((_B, _L, _H), jnp.float32),
        scratch_types=[
            pltpu.VMEM((_CH, _H), jnp.float32),
            pltpu.SemaphoreType.DMA,
        ],
    )
    def body(table_hbm, out_hbm, buf, sem):
        wid = lax.axis_index("s") * info.num_cores + lax.axis_index("c")
        base = wid * rows_per_w
        for i in range(n_ch):
            off = base + i * _CH
            pltpu.sync_copy(table_hbm.at[pl.ds(off, _CH)], buf)
            copies = [
                pltpu.async_copy(buf, out_hbm.at[b, pl.ds(off, _CH)], sem)
                for b in range(_B)
            ]
            for cp in copies:
                cp.wait()

    return body(table)


def kernel(x, table):
    del x
    return _broadcast_rows(table)
